# Initial kernel scaffold; baseline (speedup 1.0000x reference)
import jax
import jax.numpy as jnp
from jax.experimental import pallas as pl


def kernel(embedding_user, embedding_item, interaction_users, interaction_items):
    raise NotImplementedError("write your pallas kernel here")


# speedup vs baseline: 4.6890x; 4.6890x over previous
import functools

import jax
import jax.numpy as jnp
from jax import lax
from jax.experimental import pallas as pl
from jax.experimental.pallas import tpu as pltpu
from jax.experimental.pallas import tpu_sc as plsc

NU = 5000
NI = 5000
D = 128
E = 320000

NTILE = 16
NPAD = 5120
EPAD = 327680
ROWS_PT = NPAD // NTILE
CHUNK = 128
EDGES_PT = EPAD // NTILE
NCHUNK = EDGES_PT // CHUNK
DW = 128
TC_BLK = 640


def _degrees(u_idx, i_idx):
  mesh = plsc.VectorSubcoreMesh(core_axis_name="c", subcore_axis_name="s")

  @functools.partial(
      pl.kernel, mesh=mesh,
      out_type=(jax.ShapeDtypeStruct((NPAD, DW), jnp.float32),
                jax.ShapeDtypeStruct((NPAD, DW), jnp.float32)),
      scratch_types=[
          pltpu.VMEM_SHARED((NPAD, DW), jnp.float32),
          pltpu.VMEM((CHUNK,), jnp.int32),
          pltpu.VMEM((CHUNK, DW), jnp.float32),
          pltpu.VMEM((NTILE, DW), jnp.float32),
          pltpu.VMEM((ROWS_PT, DW), jnp.float32),
      ],
  )
  def k(u_hbm, i_hbm, du_hbm, di_hbm, deg_sp, idx_v, ones_v, zero_v,
        stage_v):
    c = lax.axis_index("c")
    s = lax.axis_index("s")

    for r in range(NTILE):
      for cc in range(DW // 16):
        zero_v[r, pl.ds(cc * 16, 16)] = jnp.zeros((16,), jnp.float32)

    def fill_ones(r, carry):
      for cc in range(DW // 16):
        ones_v[r, pl.ds(cc * 16, 16)] = jnp.ones((16,), jnp.float32)
      return carry
    lax.fori_loop(0, CHUNK, fill_ones, 0)

    def run(idx_hbm, out_hbm):
      def zrow(j, carry):
        pltpu.sync_copy(zero_v, deg_sp.at[pl.ds(s * ROWS_PT + j * NTILE,
                                                NTILE)])
        return carry
      lax.fori_loop(0, ROWS_PT // NTILE, zrow, 0)
      plsc.subcore_barrier()

      base0 = s * EDGES_PT
      def chunk(g, carry):
        pltpu.sync_copy(idx_hbm.at[pl.ds(base0 + g * CHUNK, CHUNK)], idx_v)
        pltpu.sync_copy(ones_v, deg_sp.at[idx_v], add=True)
        return carry
      lax.fori_loop(0, NCHUNK, chunk, 0)
      plsc.subcore_barrier()

      r0 = s * ROWS_PT
      pltpu.sync_copy(deg_sp.at[pl.ds(r0, ROWS_PT)], stage_v)
      pltpu.sync_copy(stage_v, out_hbm.at[pl.ds(r0, ROWS_PT)])

    @pl.when(c == 0)
    def _():
      run(u_hbm, du_hbm)

    @pl.when(c == 1)
    def _():
      run(i_hbm, di_hbm)

  return k(u_idx, i_idx)


def _propagate(u_idx, i_idx, ut, it):
  mesh = plsc.VectorSubcoreMesh(core_axis_name="c", subcore_axis_name="s")

  @functools.partial(
      pl.kernel, mesh=mesh,
      out_type=(jax.ShapeDtypeStruct((NPAD, D), jnp.float32),
                jax.ShapeDtypeStruct((NPAD, D), jnp.float32)),
      scratch_types=[
          pltpu.VMEM_SHARED((NPAD, D), jnp.float32),
          pltpu.VMEM((CHUNK,), jnp.int32),
          pltpu.VMEM((CHUNK,), jnp.int32),
          pltpu.VMEM((CHUNK, D), jnp.float32),
          pltpu.VMEM((NTILE, D), jnp.float32),
          pltpu.VMEM((ROWS_PT, D), jnp.float32),
          pltpu.SemaphoreType.DMA,
      ],
  )
  def k(u_hbm, i_hbm, ut_hbm, it_hbm, accu_hbm, acci_hbm,
        acc_sp, sidx_v, didx_v, rows_v, zero_v, stage_v, sem):
    c = lax.axis_index("c")
    s = lax.axis_index("s")

    for r in range(NTILE):
      for cc in range(D // 16):
        zero_v[r, pl.ds(cc * 16, 16)] = jnp.zeros((16,), jnp.float32)

    def run(src_tab, src_idx, dst_idx, out_hbm):
      def zrow(j, carry):
        pltpu.sync_copy(zero_v, acc_sp.at[pl.ds(s * ROWS_PT + j * NTILE,
                                                NTILE)])
        return carry
      lax.fori_loop(0, ROWS_PT // NTILE, zrow, 0)
      plsc.subcore_barrier()

      base0 = s * EDGES_PT
      def chunk(g, carry):
        b = base0 + g * CHUNK
        pltpu.sync_copy(src_idx.at[pl.ds(b, CHUNK)], sidx_v)
        pltpu.sync_copy(dst_idx.at[pl.ds(b, CHUNK)], didx_v)
        pltpu.async_copy(src_tab.at[sidx_v], rows_v, sem).wait()
        pltpu.sync_copy(rows_v, acc_sp.at[didx_v], add=True)
        return carry
      lax.fori_loop(0, NCHUNK, chunk, 0)
      plsc.subcore_barrier()

      r0 = s * ROWS_PT
      pltpu.sync_copy(acc_sp.at[pl.ds(r0, ROWS_PT)], stage_v)
      pltpu.sync_copy(stage_v, out_hbm.at[pl.ds(r0, ROWS_PT)])

    @pl.when(c == 0)
    def _():
      run(it_hbm, i_hbm, u_hbm, accu_hbm)

    @pl.when(c == 1)
    def _():
      run(ut_hbm, u_hbm, i_hbm, acci_hbm)

  return k(u_idx, i_idx, ut, it)


def _isd(deg_ref):
  return lax.rsqrt(jnp.maximum(deg_ref[:, 0:1], 1.0))


def _tc_scale(deg2d, tab, power):
  def body(deg_ref, tab_ref, o_ref):
    r = _isd(deg_ref)
    if power == 2:
      r = r * r
    o_ref[...] = tab_ref[...] * r

  return pl.pallas_call(
      body,
      grid=(NPAD // TC_BLK,),
      in_specs=[pl.BlockSpec((TC_BLK, DW), lambda g: (g, 0)),
                pl.BlockSpec((TC_BLK, D), lambda g: (g, 0))],
      out_specs=pl.BlockSpec((TC_BLK, D), lambda g: (g, 0)),
      out_shape=jax.ShapeDtypeStruct((NPAD, D), jnp.float32),
  )(deg2d, tab)


def _tc_final(deg2d, e0, a1, a2):
  def body(deg_ref, e0_ref, a1_ref, a2_ref, o_ref):
    r = _isd(deg_ref)
    o_ref[...] = (e0_ref[...] + (a1_ref[...] + a2_ref[...]) * r) * (1.0 / 3.0)

  return pl.pallas_call(
      body,
      grid=(NPAD // TC_BLK,),
      in_specs=[pl.BlockSpec((TC_BLK, DW), lambda g: (g, 0)),
                pl.BlockSpec((TC_BLK, D), lambda g: (g, 0)),
                pl.BlockSpec((TC_BLK, D), lambda g: (g, 0)),
                pl.BlockSpec((TC_BLK, D), lambda g: (g, 0))],
      out_specs=pl.BlockSpec((TC_BLK, D), lambda g: (g, 0)),
      out_shape=jax.ShapeDtypeStruct((NPAD, D), jnp.float32),
  )(deg2d, e0, a1, a2)


def kernel(embedding_user, embedding_item, interaction_users,
           interaction_items):
  u0p = jnp.pad(embedding_user, ((0, NPAD - NU), (0, 0)))
  i0p = jnp.pad(embedding_item, ((0, NPAD - NI), (0, 0)))
  pad_idx = jnp.full((EPAD - E,), NPAD - 1, jnp.int32)
  up = jnp.concatenate([interaction_users, pad_idx])
  ip = jnp.concatenate([interaction_items, pad_idx])

  dg_u, dg_i = _degrees(up, ip)

  ut0 = _tc_scale(dg_u, u0p, 1)
  it0 = _tc_scale(dg_i, i0p, 1)
  acc_u1, acc_i1 = _propagate(up, ip, ut0, it0)

  ut1 = _tc_scale(dg_u, acc_u1, 2)
  it1 = _tc_scale(dg_i, acc_i1, 2)
  acc_u2, acc_i2 = _propagate(up, ip, ut1, it1)

  user_final = _tc_final(dg_u, u0p, acc_u1, acc_u2)
  item_final = _tc_final(dg_i, i0p, acc_i1, acc_i2)
  return user_final[:NU], item_final[:NI]
